# Initial kernel scaffold; baseline (speedup 1.0000x reference)
import jax
import jax.numpy as jnp
from jax.experimental import pallas as pl


def kernel(pred_logits, pred_boxes, priors, target_sizes):
    raise NotImplementedError("write your pallas kernel here")


# speedup vs baseline: 243.3754x; 243.3754x over previous
import jax
import jax.numpy as jnp
from jax import lax
from jax.experimental import pallas as pl
from jax.experimental.pallas import tpu as pltpu

_VAR0 = 0.1
_VAR1 = 0.2
_SCORE_T = 0.5
_NMS_T = 0.45
_DETS = 100
_MIN_SZ = 0.01
_NEG = float("-inf")


def _cand_kernel(scale_ref, logits_ref, rx_ref, ry_ref, rw_ref, rh_ref,
                 px_ref, py_ref, pw_ref, ph_ref,
                 score_ref, label_ref, x1_ref, y1_ref, x2_ref, y2_ref):
    lg = logits_ref[...]
    m = jnp.max(lg, axis=-1, keepdims=True)
    e = jnp.exp(lg - m)
    s = jnp.sum(e, axis=-1)
    cls = lax.broadcasted_iota(jnp.int32, lg.shape, 2)
    lg_m = jnp.where(cls >= 1, lg, _NEG)
    lmax = jnp.max(lg_m, axis=-1, keepdims=True)
    score = jnp.exp(lmax - m)[..., 0] / s
    label = jnp.min(jnp.where(lg_m == lmax, cls, jnp.int32(1 << 30)), axis=-1)

    rx = rx_ref[0]
    ry = ry_ref[0]
    rw = rw_ref[0]
    rh = rh_ref[0]
    px = px_ref[0]
    py = py_ref[0]
    pw = pw_ref[0]
    ph = ph_ref[0]
    cx = px + rx * _VAR0 * pw
    cy = py + ry * _VAR0 * ph
    w = pw * jnp.exp(rw * _VAR1)
    h = ph * jnp.exp(rh * _VAR1)
    tx = cx - w * 0.5
    ty = cy - h * 0.5
    bx = tx + w
    by = ty + h
    sx = scale_ref[:, 0:1]
    sy = scale_ref[:, 1:2]
    x1 = tx * sx
    y1 = ty * sy
    x2 = bx * sx
    y2 = by * sy
    ws = x2 - x1
    hs = y2 - y1
    valid = (score > _SCORE_T) & (ws >= _MIN_SZ) & (hs >= _MIN_SZ)
    score_ref[0] = jnp.where(valid, score, _NEG)
    label_ref[0] = label
    x1_ref[0] = x1
    y1_ref[0] = y1
    x2_ref[0] = x2
    y2_ref[0] = y2


def _nms_kernel(score_ref, label_ref, x1_ref, y1_ref, x2_ref, y2_ref,
                obox_ref, osc_ref, olab_ref,
                act_ref, cx1_ref, cy1_ref, cx2_ref, cy2_ref, area_ref,
                maxc_ref):
    G, B, NB = score_ref.shape
    sc = score_ref[...]
    validm = sc > _NEG
    lab_f = label_ref[...].astype(jnp.float32)

    def red(a, fn, keep_id):
        r = fn(jnp.where(validm, a, keep_id) if keep_id is not None else a,
               axis=2, keepdims=True)
        return fn(r, axis=0, keepdims=True)

    x1 = x1_ref[...]
    y1 = y1_ref[...]
    x2 = x2_ref[...]
    y2 = y2_ref[...]
    maxc = jnp.maximum(jnp.maximum(red(x1, jnp.max, _NEG), red(y1, jnp.max, _NEG)),
                       jnp.maximum(red(x2, jnp.max, _NEG), red(y2, jnp.max, _NEG)))
    off = lab_f * (maxc + 1.0)
    cx1 = x1 + off
    cy1 = y1 + off
    cx2 = x2 + off
    cy2 = y2 + off
    act_ref[...] = sc
    cx1_ref[...] = cx1
    cy1_ref[...] = cy1
    cx2_ref[...] = cx2
    cy2_ref[...] = cy2
    area_ref[...] = (cx2 - cx1) * (cy2 - cy1)
    maxc_ref[...] = maxc

    ig = lax.broadcasted_iota(jnp.int32, (G, B, NB), 0)
    ik = lax.broadcasted_iota(jnp.int32, (G, B, NB), 2)
    iota = ig * NB + ik

    def rmax(a):
        return jnp.max(jnp.max(a, axis=2, keepdims=True), axis=0, keepdims=True)

    def rmin(a):
        return jnp.min(jnp.min(a, axis=2, keepdims=True), axis=0, keepdims=True)

    def rsum(a):
        return jnp.sum(jnp.sum(a, axis=2, keepdims=True), axis=0, keepdims=True)

    def body(t, carry):
        a = act_ref[...]
        m = rmax(a)
        has = m > _NEG
        selm = a == m
        idx = rmin(jnp.where(selm, iota, jnp.int32(1 << 30)))
        sel = iota == idx

        def ext(arr):
            return rsum(jnp.where(sel, arr, 0.0))

        sx1 = ext(x1_ref[...])
        sy1 = ext(y1_ref[...])
        sx2 = ext(x2_ref[...])
        sy2 = ext(y2_ref[...])
        slab = rsum(jnp.where(sel, label_ref[...], 0))
        soff = slab.astype(jnp.float32) * (maxc_ref[...] + 1.0)
        scx1 = sx1 + soff
        scy1 = sy1 + soff
        scx2 = sx2 + soff
        scy2 = sy2 + soff
        sarea = (scx2 - scx1) * (scy2 - scy1)

        xx1 = jnp.maximum(scx1, cx1_ref[...])
        yy1 = jnp.maximum(scy1, cy1_ref[...])
        xx2 = jnp.minimum(scx2, cx2_ref[...])
        yy2 = jnp.minimum(scy2, cy2_ref[...])
        w = jnp.maximum(0.0, xx2 - xx1)
        h = jnp.maximum(0.0, yy2 - yy1)
        inter = w * h
        iou = inter / (sarea + area_ref[...] - inter + 1e-12)
        supp = (iou > _NMS_T) | sel
        act_ref[...] = jnp.where(has & supp, _NEG, a)

        zf = jnp.float32(0.0)
        ob = jnp.concatenate(
            [jnp.where(has, sx1, zf), jnp.where(has, sy1, zf),
             jnp.where(has, sx2, zf), jnp.where(has, sy2, zf)], axis=2)
        obox_ref[pl.ds(t, 1)] = ob[0][None]
        osc_ref[pl.ds(t, 1)] = jnp.where(has, m, zf)[0][None]
        olab_ref[pl.ds(t, 1)] = jnp.where(has, slab, 0)[0][None]
        return carry

    lax.fori_loop(0, _DETS, body, 0)


def kernel(pred_logits, pred_boxes, priors, target_sizes):
    B, N, C = pred_logits.shape
    NB = 1000
    G = N // NB
    ts = target_sizes.astype(jnp.float32)
    scale = jnp.stack([ts[:, 1], ts[:, 0]], axis=1)

    def coord(a, k):
        return a[:, :, k].reshape(B, G, NB).transpose(1, 0, 2)

    def pcoord(k):
        return priors[:, k].reshape(G, 1, NB)

    rx, ry, rw, rh = (coord(pred_boxes, k) for k in range(4))
    px, py, pw, ph = (pcoord(k) for k in range(4))

    fld = jax.ShapeDtypeStruct((G, B, NB), jnp.float32)
    gbn = pl.BlockSpec((1, B, NB), lambda i: (i, 0, 0))
    g1n = pl.BlockSpec((1, 1, NB), lambda i: (i, 0, 0))
    score, label, x1, y1, x2, y2 = pl.pallas_call(
        _cand_kernel,
        grid=(G,),
        in_specs=[
            pl.BlockSpec((B, 2), lambda i: (0, 0)),
            pl.BlockSpec((B, NB, C), lambda i: (0, i, 0)),
            gbn, gbn, gbn, gbn, g1n, g1n, g1n, g1n,
        ],
        out_specs=[pl.BlockSpec((1, B, NB), lambda i: (i, 0, 0))] * 6,
        out_shape=[fld, jax.ShapeDtypeStruct((G, B, NB), jnp.int32),
                   fld, fld, fld, fld],
    )(scale, pred_logits, rx, ry, rw, rh, px, py, pw, ph)

    obox, osc, olab = pl.pallas_call(
        _nms_kernel,
        out_shape=[
            jax.ShapeDtypeStruct((_DETS, B, 4), jnp.float32),
            jax.ShapeDtypeStruct((_DETS, B, 1), jnp.float32),
            jax.ShapeDtypeStruct((_DETS, B, 1), jnp.int32),
        ],
        scratch_shapes=[pltpu.VMEM((G, B, NB), jnp.float32)] * 6
        + [pltpu.VMEM((1, B, 1), jnp.float32)],
    )(score, label, x1, y1, x2, y2)

    out_box = jnp.transpose(obox, (1, 0, 2))
    out_score = jnp.transpose(osc[..., 0], (1, 0))
    out_label = jnp.transpose(olab[..., 0], (1, 0))
    combined = jnp.concatenate([out_box, out_score[..., None]], axis=-1)
    return (combined, out_label)


# speedup vs baseline: 290.6506x; 1.1942x over previous
import functools

import jax
import jax.numpy as jnp
from jax import lax
from jax.experimental import pallas as pl
from jax.experimental.pallas import tpu as pltpu
from jax.experimental.pallas import tpu_sc as plsc

_VAR0 = 0.1
_VAR1 = 0.2
_SCORE_T = 0.5
_NMS_T = 0.45
_DETS = 100
_MIN_SZ = 0.01
_NEG = float("-inf")

_NP = 20480
_NVEC = _NP // 16
_NGRP = _NVEC // 16
_OUTW = 112


def _cand_kernel(scale_ref, logits_ref, rx_ref, ry_ref, rw_ref, rh_ref,
                 px_ref, py_ref, pw_ref, ph_ref,
                 score_ref, label_ref, x1_ref, y1_ref, x2_ref, y2_ref):
    lg = logits_ref[...]
    m = jnp.max(lg, axis=-1, keepdims=True)
    e = jnp.exp(lg - m)
    s = jnp.sum(e, axis=-1)
    cls = lax.broadcasted_iota(jnp.int32, lg.shape, 2)
    lg_m = jnp.where(cls >= 1, lg, _NEG)
    lmax = jnp.max(lg_m, axis=-1, keepdims=True)
    score = jnp.exp(lmax - m)[..., 0] / s
    label = jnp.min(jnp.where(lg_m == lmax, cls, jnp.int32(1 << 30)),
                    axis=-1).astype(jnp.float32)

    rx = rx_ref[0]
    ry = ry_ref[0]
    rw = rw_ref[0]
    rh = rh_ref[0]
    px = px_ref[0]
    py = py_ref[0]
    pw = pw_ref[0]
    ph = ph_ref[0]
    cx = px + rx * _VAR0 * pw
    cy = py + ry * _VAR0 * ph
    w = pw * jnp.exp(rw * _VAR1)
    h = ph * jnp.exp(rh * _VAR1)
    tx = cx - w * 0.5
    ty = cy - h * 0.5
    bx = tx + w
    by = ty + h
    sx = scale_ref[:, 0:1]
    sy = scale_ref[:, 1:2]
    x1 = tx * sx
    y1 = ty * sy
    x2 = bx * sx
    y2 = by * sy
    ws = x2 - x1
    hs = y2 - y1
    valid = (score > _SCORE_T) & (ws >= _MIN_SZ) & (hs >= _MIN_SZ)
    score_ref[0] = jnp.where(valid, score, _NEG)
    label_ref[0] = label
    x1_ref[0] = x1
    y1_ref[0] = y1
    x2_ref[0] = x2
    y2_ref[0] = y2


_STAGES = (160, 352, 1536, 8192, _NP)


def _sc_nms(score_hbm, lab_hbm, x1_hbm, y1_hbm, x2_hbm, y2_hbm, out_hbm,
            score_v, lab_v, x1_v, y1_v, x2_v, y2_v,
            bm_v, bm2_v, kx1_v, ky1_v, kx2_v, ky2_v, karea_v, out_v):
    nc = 2
    wid = lax.axis_index("s") * nc + lax.axis_index("c")
    b = wid

    @pl.when(wid < 8)
    def _():
        pltpu.sync_copy(score_hbm.at[pl.ds(b * _NP, _NP)], score_v)
        pltpu.sync_copy(lab_hbm.at[pl.ds(b * _NP, _NP)], lab_v)
        pltpu.sync_copy(x1_hbm.at[pl.ds(b * _NP, _NP)], x1_v)
        pltpu.sync_copy(y1_hbm.at[pl.ds(b * _NP, _NP)], y1_v)
        pltpu.sync_copy(x2_hbm.at[pl.ds(b * _NP, _NP)], x2_v)
        pltpu.sync_copy(y2_hbm.at[pl.ds(b * _NP, _NP)], y2_v)

        li = lax.iota(jnp.int32, 16)
        negv = jnp.full((16,), _NEG, jnp.float32)
        zerov = jnp.zeros((16,), jnp.float32)

        def smax(v):
            for k in (8, 4, 2, 1):
                v = jnp.maximum(v, v[li ^ k])
            return v

        def sminv(v):
            for k in (8, 4, 2, 1):
                v = jnp.minimum(v, v[li ^ k])
            return v

        def ffsv(mask):
            return sminv(jnp.where(mask, li, jnp.int32(16)))

        def ffs(mask):
            return ffsv(mask)[0]

        for q in range(6 * _OUTW // 16):
            out_v[pl.ds(q * 16, 16)] = zerov

        bm2_v[pl.ds(_NGRP, 16)] = negv

        def group_body(g, carry):
            acc2, mx = carry
            acc16 = negv
            mxv = mx
            for jj in range(16):
                off = g * 256 + jj * 16
                sv = score_v[pl.ds(off, 16)]
                acc16 = jnp.where(li == jj, smax(sv), acc16)
                vm = sv > _NEG
                mxv = jnp.maximum(mxv, jnp.where(vm, x1_v[pl.ds(off, 16)], _NEG))
                mxv = jnp.maximum(mxv, jnp.where(vm, y1_v[pl.ds(off, 16)], _NEG))
                mxv = jnp.maximum(mxv, jnp.where(vm, x2_v[pl.ds(off, 16)], _NEG))
                mxv = jnp.maximum(mxv, jnp.where(vm, y2_v[pl.ds(off, 16)], _NEG))
            bm_v[pl.ds(g * 16, 16)] = acc16
            acc2 = jnp.where(li == (g % 16), smax(acc16), acc2)
            return acc2, mxv

        mx = negv
        for t in range(_NGRP // 16):
            acc2, mx = lax.fori_loop(t * 16, (t + 1) * 16, group_body,
                                     (negv, mx))
            bm2_v[pl.ds(t * 16, 16)] = acc2

        kcoef = smax(mx)[0] + 1.0

        def global_max():
            g = bm2_v[pl.ds(0, 16)]
            for t in range(1, 6):
                g = jnp.maximum(g, bm2_v[pl.ds(t * 16, 16)])
            return smax(g)[0]

        def body(_, carry):
            kc, gmax = carry
            alive = (kc < _DETS) & (gmax > _NEG)
            bigi = jnp.int32(1 << 30)
            g2s = bigi
            for t in range(6):
                v = bm2_v[pl.ds(t * 16, 16)]
                f = ffs(v == gmax)
                hit = f < 16
                g2s = jnp.where((g2s == bigi) & hit, t * 16 + f, g2s)
            g2 = jnp.where(alive, g2s, 0)
            bv = bm_v[pl.ds(g2 * 16, 16)]
            jfv = ffsv(bv == gmax)
            jf = jnp.where(alive, jfv[0], 0)
            j = g2 * 16 + jf
            sv = score_v[pl.ds(j * 16, 16)]
            lfv = ffsv(sv == gmax)
            lf = jnp.where(alive, lfv[0], 0)
            lmask = li == lf

            gidx = jnp.where(alive, lfv, 0)

            def extract(buf):
                return buf[pl.ds(j * 16, 16)][gidx][0]

            xv1 = extract(x1_v)
            yv1 = extract(y1_v)
            xv2 = extract(x2_v)
            yv2 = extract(y2_v)
            lv = extract(lab_v)

            sv2 = jnp.where(lmask & alive, _NEG, sv)
            score_v[pl.ds(j * 16, 16)] = sv2
            bv2 = jnp.where((li == jf) & alive, smax(sv2), bv)
            bm_v[pl.ds(g2 * 16, 16)] = bv2
            t2 = g2 // 16
            b2v = bm2_v[pl.ds(t2 * 16, 16)]
            b2v2 = jnp.where((li == (g2 % 16)) & alive, smax(bv2), b2v)
            bm2_v[pl.ds(t2 * 16, 16)] = b2v2

            offs = lv * kcoef
            cx1 = xv1 + offs
            cy1 = yv1 + offs
            cx2 = xv2 + offs
            cy2 = yv2 + offs
            carea = (cx2 - cx1) * (cy2 - cy1)

            def kbody(q, rej):
                lanev = q * 16 + li
                xx1 = jnp.maximum(cx1, kx1_v[pl.ds(q * 16, 16)])
                yy1 = jnp.maximum(cy1, ky1_v[pl.ds(q * 16, 16)])
                xx2 = jnp.minimum(cx2, kx2_v[pl.ds(q * 16, 16)])
                yy2 = jnp.minimum(cy2, ky2_v[pl.ds(q * 16, 16)])
                w = jnp.maximum(0.0, xx2 - xx1)
                h = jnp.maximum(0.0, yy2 - yy1)
                inter = w * h
                iou = inter / (carea + karea_v[pl.ds(q * 16, 16)] - inter
                               + 1e-12)
                m = (iou > _NMS_T) & (lanev < kc)
                return rej | jnp.where(ffs(m) < 16, jnp.int32(1), jnp.int32(0))

            nk = (kc + 15) // 16
            rej = lax.fori_loop(0, nk, kbody, jnp.int32(0))
            acc = (rej == 0) & alive

            q = kc // 16
            l = kc % 16
            am = acc & (li == l)

            def ins(buf, base, val):
                cur = buf[pl.ds(base + q * 16, 16)]
                buf[pl.ds(base + q * 16, 16)] = jnp.where(am, val, cur)

            ins(kx1_v, 0, cx1)
            ins(ky1_v, 0, cy1)
            ins(kx2_v, 0, cx2)
            ins(ky2_v, 0, cy2)
            ins(karea_v, 0, carea)
            ins(out_v, 0 * _OUTW, xv1)
            ins(out_v, 1 * _OUTW, yv1)
            ins(out_v, 2 * _OUTW, xv2)
            ins(out_v, 3 * _OUTW, yv2)
            ins(out_v, 4 * _OUTW, gmax)
            ins(out_v, 5 * _OUTW, lv)

            kc2 = kc + jnp.where(acc, 1, 0)
            return kc2, global_max()

        carry = (jnp.int32(0), global_max())
        for cap in _STAGES:
            kc, gmax = carry
            n = jnp.where((kc < _DETS) & (gmax > _NEG), cap, 0)
            carry = lax.fori_loop(0, n, body, carry)
        pltpu.sync_copy(out_v, out_hbm.at[pl.ds(b * (6 * _OUTW), 6 * _OUTW)])


def kernel(pred_logits, pred_boxes, priors, target_sizes):
    B, N, C = pred_logits.shape
    NB = 1000
    G = N // NB
    ts = target_sizes.astype(jnp.float32)
    scale = jnp.stack([ts[:, 1], ts[:, 0]], axis=1)

    def coord(a, k):
        return a[:, :, k].reshape(B, G, NB).transpose(1, 0, 2)

    def pcoord(k):
        return priors[:, k].reshape(G, 1, NB)

    rx, ry, rw, rh = (coord(pred_boxes, k) for k in range(4))
    px, py, pw, ph = (pcoord(k) for k in range(4))

    fld = jax.ShapeDtypeStruct((G, B, NB), jnp.float32)
    gbn = pl.BlockSpec((1, B, NB), lambda i: (i, 0, 0))
    g1n = pl.BlockSpec((1, 1, NB), lambda i: (i, 0, 0))
    score, label, x1, y1, x2, y2 = pl.pallas_call(
        _cand_kernel,
        grid=(G,),
        in_specs=[
            pl.BlockSpec((B, 2), lambda i: (0, 0)),
            pl.BlockSpec((B, NB, C), lambda i: (0, i, 0)),
            gbn, gbn, gbn, gbn, g1n, g1n, g1n, g1n,
        ],
        out_specs=[pl.BlockSpec((1, B, NB), lambda i: (i, 0, 0))] * 6,
        out_shape=[fld, fld, fld, fld, fld, fld],
    )(scale, pred_logits, rx, ry, rw, rh, px, py, pw, ph)

    def to8(a, pad):
        t = a.transpose(1, 0, 2).reshape(B, N)
        return jnp.pad(t, ((0, 0), (0, _NP - N)),
                       constant_values=pad).reshape(B * _NP)

    score8 = to8(score, _NEG)
    lab8 = to8(label, 0.0)
    x18 = to8(x1, 0.0)
    y18 = to8(y1, 0.0)
    x28 = to8(x2, 0.0)
    y28 = to8(y2, 0.0)

    mesh = plsc.VectorSubcoreMesh(core_axis_name="c", subcore_axis_name="s")
    big = pltpu.VMEM((_NP,), jnp.float32)
    kept = pltpu.VMEM((_OUTW,), jnp.float32)
    out = functools.partial(
        pl.kernel,
        mesh=mesh,
        compiler_params=pltpu.CompilerParams(needs_layout_passes=False),
        out_type=jax.ShapeDtypeStruct((B * 6 * _OUTW,), jnp.float32),
        scratch_types=[big, big, big, big, big, big,
                       pltpu.VMEM((_NVEC,), jnp.float32),
                       pltpu.VMEM((96,), jnp.float32),
                       kept, kept, kept, kept, kept,
                       pltpu.VMEM((6 * _OUTW,), jnp.float32)],
    )(_sc_nms)(score8, lab8, x18, y18, x28, y28)

    o = out.reshape(B, 6, _OUTW)[:, :, :_DETS]
    combined = jnp.stack([o[:, 0], o[:, 1], o[:, 2], o[:, 3], o[:, 4]],
                         axis=-1)
    labels = o[:, 5].astype(jnp.int32)
    return (combined, labels)


# speedup vs baseline: 310.7648x; 1.0692x over previous
import functools

import jax
import jax.numpy as jnp
from jax import lax
from jax.experimental import pallas as pl
from jax.experimental.pallas import tpu as pltpu
from jax.experimental.pallas import tpu_sc as plsc

_VAR0 = 0.1
_VAR1 = 0.2
_SCORE_T = 0.5
_NMS_T = 0.45
_DETS = 100
_MIN_SZ = 0.01
_NEG = float("-inf")

_NP = 20480
_NVEC = _NP // 16
_NGRP = _NVEC // 16
_OUTW = 112


def _cand_kernel(scale_ref, clsw_ref, mask0_ref, logits_ref, rx_ref, ry_ref,
                 rw_ref, rh_ref, px_ref, py_ref, pw_ref, ph_ref,
                 score_ref, label_ref, x1_ref, y1_ref, x2_ref, y2_ref):
    lg = logits_ref[...]
    m = jnp.max(lg, axis=-1, keepdims=True)
    e = jnp.exp(lg - m)
    s = jnp.sum(e, axis=-1)
    e80 = e * mask0_ref[...]
    emax = jnp.max(e80, axis=-1, keepdims=True)
    score = emax[..., 0] / s
    label = jnp.max(jnp.where(e80 == emax, clsw_ref[...], 0.0), axis=-1)

    rx = rx_ref[0]
    ry = ry_ref[0]
    rw = rw_ref[0]
    rh = rh_ref[0]
    px = px_ref[0]
    py = py_ref[0]
    pw = pw_ref[0]
    ph = ph_ref[0]
    cx = px + rx * _VAR0 * pw
    cy = py + ry * _VAR0 * ph
    w = pw * jnp.exp(rw * _VAR1)
    h = ph * jnp.exp(rh * _VAR1)
    tx = cx - w * 0.5
    ty = cy - h * 0.5
    bx = tx + w
    by = ty + h
    sx = scale_ref[:, 0:1]
    sy = scale_ref[:, 1:2]
    x1 = tx * sx
    y1 = ty * sy
    x2 = bx * sx
    y2 = by * sy
    ws = x2 - x1
    hs = y2 - y1
    valid = (score > _SCORE_T) & (ws >= _MIN_SZ) & (hs >= _MIN_SZ)
    score_ref[0] = jnp.where(valid, score, _NEG)
    label_ref[0] = label
    x1_ref[0] = x1
    y1_ref[0] = y1
    x2_ref[0] = x2
    y2_ref[0] = y2


_STAGES = (160, 352, 1536, 8192, _NP)


def _sc_nms(score_hbm, lab_hbm, x1_hbm, y1_hbm, x2_hbm, y2_hbm, out_hbm,
            score_v, lab_v, x1_v, y1_v, x2_v, y2_v,
            bm_v, bm2_v, kx1_v, ky1_v, kx2_v, ky2_v, karea_v, out_v):
    nc = 2
    wid = lax.axis_index("s") * nc + lax.axis_index("c")
    b = wid

    @pl.when(wid < 8)
    def _():
        pltpu.sync_copy(score_hbm.at[pl.ds(b * _NP, _NP)], score_v)
        pltpu.sync_copy(lab_hbm.at[pl.ds(b * _NP, _NP)], lab_v)
        pltpu.sync_copy(x1_hbm.at[pl.ds(b * _NP, _NP)], x1_v)
        pltpu.sync_copy(y1_hbm.at[pl.ds(b * _NP, _NP)], y1_v)
        pltpu.sync_copy(x2_hbm.at[pl.ds(b * _NP, _NP)], x2_v)
        pltpu.sync_copy(y2_hbm.at[pl.ds(b * _NP, _NP)], y2_v)

        li = lax.iota(jnp.int32, 16)
        negv = jnp.full((16,), _NEG, jnp.float32)
        zerov = jnp.zeros((16,), jnp.float32)

        def smax(v):
            for k in (8, 4, 2, 1):
                v = jnp.maximum(v, v[li ^ k])
            return v

        def sminv(v):
            for k in (8, 4, 2, 1):
                v = jnp.minimum(v, v[li ^ k])
            return v

        def ffsv(mask):
            return sminv(jnp.where(mask, li, jnp.int32(16)))

        def ffs(mask):
            return ffsv(mask)[0]

        for q in range(6 * _OUTW // 16):
            out_v[pl.ds(q * 16, 16)] = zerov

        bm2_v[pl.ds(_NGRP, 16)] = negv

        def group_body(g, carry):
            acc2, mx = carry
            acc16 = negv
            mxv = mx
            for jj in range(16):
                off = g * 256 + jj * 16
                sv = score_v[pl.ds(off, 16)]
                acc16 = jnp.where(li == jj, smax(sv), acc16)
                vm = sv > _NEG
                mxv = jnp.maximum(mxv, jnp.where(vm, x1_v[pl.ds(off, 16)], _NEG))
                mxv = jnp.maximum(mxv, jnp.where(vm, y1_v[pl.ds(off, 16)], _NEG))
                mxv = jnp.maximum(mxv, jnp.where(vm, x2_v[pl.ds(off, 16)], _NEG))
                mxv = jnp.maximum(mxv, jnp.where(vm, y2_v[pl.ds(off, 16)], _NEG))
            bm_v[pl.ds(g * 16, 16)] = acc16
            acc2 = jnp.where(li == (g % 16), smax(acc16), acc2)
            return acc2, mxv

        mx = negv
        for t in range(_NGRP // 16):
            acc2, mx = lax.fori_loop(t * 16, (t + 1) * 16, group_body,
                                     (negv, mx))
            bm2_v[pl.ds(t * 16, 16)] = acc2

        kcoef = smax(mx)[0] + 1.0

        def global_max():
            g = bm2_v[pl.ds(0, 16)]
            for t in range(1, 6):
                g = jnp.maximum(g, bm2_v[pl.ds(t * 16, 16)])
            return smax(g)[0]

        def body(_, carry):
            kc, gmax = carry
            alive = (kc < _DETS) & (gmax > _NEG)
            bigi = jnp.int32(1 << 30)
            g2s = bigi
            for t in range(6):
                v = bm2_v[pl.ds(t * 16, 16)]
                f = ffs(v == gmax)
                hit = f < 16
                g2s = jnp.where((g2s == bigi) & hit, t * 16 + f, g2s)
            g2 = jnp.where(alive, g2s, 0)
            bv = bm_v[pl.ds(g2 * 16, 16)]
            jfv = ffsv(bv == gmax)
            jf = jnp.where(alive, jfv[0], 0)
            j = g2 * 16 + jf
            sv = score_v[pl.ds(j * 16, 16)]
            lfv = ffsv(sv == gmax)
            lf = jnp.where(alive, lfv[0], 0)
            lmask = li == lf

            gidx = jnp.where(alive, lfv, 0)

            def extract(buf):
                return buf[pl.ds(j * 16, 16)][gidx][0]

            xv1 = extract(x1_v)
            yv1 = extract(y1_v)
            xv2 = extract(x2_v)
            yv2 = extract(y2_v)
            lv = extract(lab_v)

            sv2 = jnp.where(lmask & alive, _NEG, sv)
            score_v[pl.ds(j * 16, 16)] = sv2
            bv2 = jnp.where((li == jf) & alive, smax(sv2), bv)
            bm_v[pl.ds(g2 * 16, 16)] = bv2
            t2 = g2 // 16
            b2v = bm2_v[pl.ds(t2 * 16, 16)]
            b2v2 = jnp.where((li == (g2 % 16)) & alive, smax(bv2), b2v)
            bm2_v[pl.ds(t2 * 16, 16)] = b2v2

            offs = lv * kcoef
            cx1 = xv1 + offs
            cy1 = yv1 + offs
            cx2 = xv2 + offs
            cy2 = yv2 + offs
            carea = (cx2 - cx1) * (cy2 - cy1)

            def kbody(q, rej):
                lanev = q * 16 + li
                xx1 = jnp.maximum(cx1, kx1_v[pl.ds(q * 16, 16)])
                yy1 = jnp.maximum(cy1, ky1_v[pl.ds(q * 16, 16)])
                xx2 = jnp.minimum(cx2, kx2_v[pl.ds(q * 16, 16)])
                yy2 = jnp.minimum(cy2, ky2_v[pl.ds(q * 16, 16)])
                w = jnp.maximum(0.0, xx2 - xx1)
                h = jnp.maximum(0.0, yy2 - yy1)
                inter = w * h
                iou = inter / (carea + karea_v[pl.ds(q * 16, 16)] - inter
                               + 1e-12)
                m = (iou > _NMS_T) & (lanev < kc)
                return rej | jnp.where(ffs(m) < 16, jnp.int32(1), jnp.int32(0))

            nk = (kc + 15) // 16
            rej = lax.fori_loop(0, nk, kbody, jnp.int32(0))
            acc = (rej == 0) & alive

            q = kc // 16
            l = kc % 16
            am = acc & (li == l)

            def ins(buf, base, val):
                cur = buf[pl.ds(base + q * 16, 16)]
                buf[pl.ds(base + q * 16, 16)] = jnp.where(am, val, cur)

            ins(kx1_v, 0, cx1)
            ins(ky1_v, 0, cy1)
            ins(kx2_v, 0, cx2)
            ins(ky2_v, 0, cy2)
            ins(karea_v, 0, carea)
            ins(out_v, 0 * _OUTW, xv1)
            ins(out_v, 1 * _OUTW, yv1)
            ins(out_v, 2 * _OUTW, xv2)
            ins(out_v, 3 * _OUTW, yv2)
            ins(out_v, 4 * _OUTW, gmax)
            ins(out_v, 5 * _OUTW, lv)

            kc2 = kc + jnp.where(acc, 1, 0)
            return kc2, global_max()

        carry = (jnp.int32(0), global_max())
        for cap in _STAGES:
            kc, gmax = carry
            n = jnp.where((kc < _DETS) & (gmax > _NEG), cap, 0)
            carry = lax.fori_loop(0, n, body, carry)
        pltpu.sync_copy(out_v, out_hbm.at[pl.ds(b * (6 * _OUTW), 6 * _OUTW)])


def kernel(pred_logits, pred_boxes, priors, target_sizes):
    B, N, C = pred_logits.shape
    NB = 1000
    G = N // NB
    ts = target_sizes.astype(jnp.float32)
    scale = jnp.stack([ts[:, 1], ts[:, 0]], axis=1)

    def coord(a, k):
        return a[:, :, k].reshape(B, G, NB).transpose(1, 0, 2)

    def pcoord(k):
        return priors[:, k].reshape(G, 1, NB)

    rx, ry, rw, rh = (coord(pred_boxes, k) for k in range(4))
    px, py, pw, ph = (pcoord(k) for k in range(4))

    clsw = jnp.arange(0, C, dtype=jnp.float32).reshape(1, 1, C)
    mask0 = jnp.concatenate([jnp.zeros((1,), jnp.float32),
                             jnp.ones((C - 1,), jnp.float32)]
                            ).reshape(1, 1, C)

    fld = jax.ShapeDtypeStruct((G, B, NB), jnp.float32)
    gbn = pl.BlockSpec((1, B, NB), lambda i: (i, 0, 0))
    g1n = pl.BlockSpec((1, 1, NB), lambda i: (i, 0, 0))
    score, label, x1, y1, x2, y2 = pl.pallas_call(
        _cand_kernel,
        grid=(G,),
        in_specs=[
            pl.BlockSpec((B, 2), lambda i: (0, 0)),
            pl.BlockSpec((1, 1, C), lambda i: (0, 0, 0)),
            pl.BlockSpec((1, 1, C), lambda i: (0, 0, 0)),
            pl.BlockSpec((B, NB, C), lambda i: (0, i, 0)),
            gbn, gbn, gbn, gbn, g1n, g1n, g1n, g1n,
        ],
        out_specs=[pl.BlockSpec((1, B, NB), lambda i: (i, 0, 0))] * 6,
        out_shape=[fld, fld, fld, fld, fld, fld],
    )(scale, clsw, mask0, pred_logits, rx, ry, rw, rh, px, py, pw, ph)

    def to8(a, pad):
        t = a.transpose(1, 0, 2).reshape(B, N)
        return jnp.pad(t, ((0, 0), (0, _NP - N)),
                       constant_values=pad).reshape(B * _NP)

    score8 = to8(score, _NEG)
    lab8 = to8(label, 0.0)
    x18 = to8(x1, 0.0)
    y18 = to8(y1, 0.0)
    x28 = to8(x2, 0.0)
    y28 = to8(y2, 0.0)

    mesh = plsc.VectorSubcoreMesh(core_axis_name="c", subcore_axis_name="s")
    big = pltpu.VMEM((_NP,), jnp.float32)
    kept = pltpu.VMEM((_OUTW,), jnp.float32)
    out = functools.partial(
        pl.kernel,
        mesh=mesh,
        compiler_params=pltpu.CompilerParams(needs_layout_passes=False),
        out_type=jax.ShapeDtypeStruct((B * 6 * _OUTW,), jnp.float32),
        scratch_types=[big, big, big, big, big, big,
                       pltpu.VMEM((_NVEC,), jnp.float32),
                       pltpu.VMEM((96,), jnp.float32),
                       kept, kept, kept, kept, kept,
                       pltpu.VMEM((6 * _OUTW,), jnp.float32)],
    )(_sc_nms)(score8, lab8, x18, y18, x28, y28)

    o = out.reshape(B, 6, _OUTW)[:, :, :_DETS]
    combined = jnp.stack([o[:, 0], o[:, 1], o[:, 2], o[:, 3], o[:, 4]],
                         axis=-1)
    labels = o[:, 5].astype(jnp.int32)
    return (combined, labels)


# speedup vs baseline: 315.1212x; 1.0140x over previous
import functools

import jax
import jax.numpy as jnp
from jax import lax
from jax.experimental import pallas as pl
from jax.experimental.pallas import tpu as pltpu
from jax.experimental.pallas import tpu_sc as plsc

_VAR0 = 0.1
_VAR1 = 0.2
_SCORE_T = 0.5
_NMS_T = 0.45
_DETS = 100
_MIN_SZ = 0.01
_NEG = float("-inf")

_N = 20000
_NB = 1000
_G = _N // _NB
_NP = 20480
_NVEC = _NP // 16
_NGRP = _NVEC // 16
_OUTW = 112


def _cand_kernel(scale_ref, clsw_ref, mask0_ref, logits_ref, rx_ref, ry_ref,
                 rw_ref, rh_ref, px_ref, py_ref, pw_ref, ph_ref,
                 score_ref, label_ref, x1_ref, y1_ref, x2_ref, y2_ref):
    lg = logits_ref[...]
    m = jnp.max(lg, axis=-1, keepdims=True)
    e = jnp.exp(lg - m)
    s = jnp.sum(e, axis=-1)
    e80 = e * mask0_ref[...]
    emax = jnp.max(e80, axis=-1, keepdims=True)
    score = emax[..., 0] / s
    label = jnp.max(jnp.where(e80 == emax, clsw_ref[...], 0.0), axis=-1)

    rx = rx_ref[0]
    ry = ry_ref[0]
    rw = rw_ref[0]
    rh = rh_ref[0]
    px = px_ref[0]
    py = py_ref[0]
    pw = pw_ref[0]
    ph = ph_ref[0]
    cx = px + rx * _VAR0 * pw
    cy = py + ry * _VAR0 * ph
    w = pw * jnp.exp(rw * _VAR1)
    h = ph * jnp.exp(rh * _VAR1)
    tx = cx - w * 0.5
    ty = cy - h * 0.5
    bx = tx + w
    by = ty + h
    sx = scale_ref[:, 0:1]
    sy = scale_ref[:, 1:2]
    x1 = tx * sx
    y1 = ty * sy
    x2 = bx * sx
    y2 = by * sy
    ws = x2 - x1
    hs = y2 - y1
    valid = (score > _SCORE_T) & (ws >= _MIN_SZ) & (hs >= _MIN_SZ)
    score_ref[0] = jnp.where(valid, score, _NEG)
    label_ref[0] = label
    x1_ref[0] = x1
    y1_ref[0] = y1
    x2_ref[0] = x2
    y2_ref[0] = y2


_STAGES = (160, 352, 1536, 8192, _NP)


def _sc_nms(score_hbm, lab_hbm, x1_hbm, y1_hbm, x2_hbm, y2_hbm, out_hbm,
            score_v, lab_v, x1_v, y1_v, x2_v, y2_v,
            bm_v, bm2_v, kx1_v, ky1_v, kx2_v, ky2_v, karea_v, out_v, sem):
    nc = 2
    wid = lax.axis_index("s") * nc + lax.axis_index("c")
    b = wid

    @pl.when(wid < 8)
    def _():
        fields = ((score_hbm, score_v), (lab_hbm, lab_v), (x1_hbm, x1_v),
                  (y1_hbm, y1_v), (x2_hbm, x2_v), (y2_hbm, y2_v))
        handles = []
        for src, dst in fields:
            for g in range(_G):
                handles.append(pltpu.async_copy(
                    src.at[pl.ds((g * 8 + b) * _NB, _NB)],
                    dst.at[pl.ds(g * _NB, _NB)], sem))
        for h in handles:
            h.wait()

        li = lax.iota(jnp.int32, 16)
        negv = jnp.full((16,), _NEG, jnp.float32)
        zerov = jnp.zeros((16,), jnp.float32)

        for q in range((_NP - _N) // 16):
            score_v[pl.ds(_N + q * 16, 16)] = negv
            lab_v[pl.ds(_N + q * 16, 16)] = zerov
            x1_v[pl.ds(_N + q * 16, 16)] = zerov
            y1_v[pl.ds(_N + q * 16, 16)] = zerov
            x2_v[pl.ds(_N + q * 16, 16)] = zerov
            y2_v[pl.ds(_N + q * 16, 16)] = zerov

        def smax(v):
            for k in (8, 4, 2, 1):
                v = jnp.maximum(v, v[li ^ k])
            return v

        def sminv(v):
            for k in (8, 4, 2, 1):
                v = jnp.minimum(v, v[li ^ k])
            return v

        def ffsv(mask):
            return sminv(jnp.where(mask, li, jnp.int32(16)))

        def ffs(mask):
            return ffsv(mask)[0]

        for q in range(6 * _OUTW // 16):
            out_v[pl.ds(q * 16, 16)] = zerov

        bm2_v[pl.ds(_NGRP, 16)] = negv

        def group_body(g, carry):
            acc2, mx = carry
            acc16 = negv
            mxv = mx
            for jj in range(16):
                off = g * 256 + jj * 16
                sv = score_v[pl.ds(off, 16)]
                acc16 = jnp.where(li == jj, smax(sv), acc16)
                vm = sv > _NEG
                mxv = jnp.maximum(mxv, jnp.where(vm, x1_v[pl.ds(off, 16)], _NEG))
                mxv = jnp.maximum(mxv, jnp.where(vm, y1_v[pl.ds(off, 16)], _NEG))
                mxv = jnp.maximum(mxv, jnp.where(vm, x2_v[pl.ds(off, 16)], _NEG))
                mxv = jnp.maximum(mxv, jnp.where(vm, y2_v[pl.ds(off, 16)], _NEG))
            bm_v[pl.ds(g * 16, 16)] = acc16
            acc2 = jnp.where(li == (g % 16), smax(acc16), acc2)
            return acc2, mxv

        mx = negv
        for t in range(_NGRP // 16):
            acc2, mx = lax.fori_loop(t * 16, (t + 1) * 16, group_body,
                                     (negv, mx))
            bm2_v[pl.ds(t * 16, 16)] = acc2

        kcoef = smax(mx)[0] + 1.0

        def global_max():
            g = bm2_v[pl.ds(0, 16)]
            for t in range(1, 6):
                g = jnp.maximum(g, bm2_v[pl.ds(t * 16, 16)])
            return smax(g)[0]

        def body(_, carry):
            kc, gmax = carry
            alive = (kc < _DETS) & (gmax > _NEG)
            bigi = jnp.int32(1 << 30)
            g2s = bigi
            for t in range(6):
                v = bm2_v[pl.ds(t * 16, 16)]
                f = ffs(v == gmax)
                hit = f < 16
                g2s = jnp.where((g2s == bigi) & hit, t * 16 + f, g2s)
            g2 = jnp.where(alive, g2s, 0)
            bv = bm_v[pl.ds(g2 * 16, 16)]
            jfv = ffsv(bv == gmax)
            jf = jnp.where(alive, jfv[0], 0)
            j = g2 * 16 + jf
            sv = score_v[pl.ds(j * 16, 16)]
            lfv = ffsv(sv == gmax)
            lf = jnp.where(alive, lfv[0], 0)
            lmask = li == lf

            gidx = jnp.where(alive, lfv, 0)

            def extract(buf):
                return buf[pl.ds(j * 16, 16)][gidx][0]

            xv1 = extract(x1_v)
            yv1 = extract(y1_v)
            xv2 = extract(x2_v)
            yv2 = extract(y2_v)
            lv = extract(lab_v)

            sv2 = jnp.where(lmask & alive, _NEG, sv)
            score_v[pl.ds(j * 16, 16)] = sv2
            bv2 = jnp.where((li == jf) & alive, smax(sv2), bv)
            bm_v[pl.ds(g2 * 16, 16)] = bv2
            t2 = g2 // 16
            b2v = bm2_v[pl.ds(t2 * 16, 16)]
            b2v2 = jnp.where((li == (g2 % 16)) & alive, smax(bv2), b2v)
            bm2_v[pl.ds(t2 * 16, 16)] = b2v2

            offs = lv * kcoef
            cx1 = xv1 + offs
            cy1 = yv1 + offs
            cx2 = xv2 + offs
            cy2 = yv2 + offs
            carea = (cx2 - cx1) * (cy2 - cy1)

            def kbody(q, rej):
                lanev = q * 16 + li
                xx1 = jnp.maximum(cx1, kx1_v[pl.ds(q * 16, 16)])
                yy1 = jnp.maximum(cy1, ky1_v[pl.ds(q * 16, 16)])
                xx2 = jnp.minimum(cx2, kx2_v[pl.ds(q * 16, 16)])
                yy2 = jnp.minimum(cy2, ky2_v[pl.ds(q * 16, 16)])
                w = jnp.maximum(0.0, xx2 - xx1)
                h = jnp.maximum(0.0, yy2 - yy1)
                inter = w * h
                iou = inter / (carea + karea_v[pl.ds(q * 16, 16)] - inter
                               + 1e-12)
                m = (iou > _NMS_T) & (lanev < kc)
                return rej | jnp.where(ffs(m) < 16, jnp.int32(1), jnp.int32(0))

            nk = (kc + 15) // 16
            rej = lax.fori_loop(0, nk, kbody, jnp.int32(0))
            acc = (rej == 0) & alive

            q = kc // 16
            l = kc % 16
            am = acc & (li == l)

            def ins(buf, base, val):
                cur = buf[pl.ds(base + q * 16, 16)]
                buf[pl.ds(base + q * 16, 16)] = jnp.where(am, val, cur)

            ins(kx1_v, 0, cx1)
            ins(ky1_v, 0, cy1)
            ins(kx2_v, 0, cx2)
            ins(ky2_v, 0, cy2)
            ins(karea_v, 0, carea)
            ins(out_v, 0 * _OUTW, xv1)
            ins(out_v, 1 * _OUTW, yv1)
            ins(out_v, 2 * _OUTW, xv2)
            ins(out_v, 3 * _OUTW, yv2)
            ins(out_v, 4 * _OUTW, gmax)
            ins(out_v, 5 * _OUTW, lv)

            kc2 = kc + jnp.where(acc, 1, 0)
            return kc2, global_max()

        carry = (jnp.int32(0), global_max())
        for cap in _STAGES:
            kc, gmax = carry
            n = jnp.where((kc < _DETS) & (gmax > _NEG), cap, 0)
            carry = lax.fori_loop(0, n, body, carry)
        pltpu.sync_copy(out_v, out_hbm.at[pl.ds(b * (6 * _OUTW), 6 * _OUTW)])


def kernel(pred_logits, pred_boxes, priors, target_sizes):
    B, N, C = pred_logits.shape
    NB = 1000
    G = N // NB
    ts = target_sizes.astype(jnp.float32)
    scale = jnp.stack([ts[:, 1], ts[:, 0]], axis=1)

    def coord(a, k):
        return a[:, :, k].reshape(B, G, NB).transpose(1, 0, 2)

    def pcoord(k):
        return priors[:, k].reshape(G, 1, NB)

    rx, ry, rw, rh = (coord(pred_boxes, k) for k in range(4))
    px, py, pw, ph = (pcoord(k) for k in range(4))

    clsw = jnp.arange(0, C, dtype=jnp.float32).reshape(1, 1, C)
    mask0 = jnp.concatenate([jnp.zeros((1,), jnp.float32),
                             jnp.ones((C - 1,), jnp.float32)]
                            ).reshape(1, 1, C)

    fld = jax.ShapeDtypeStruct((G, B, NB), jnp.float32)
    gbn = pl.BlockSpec((1, B, NB), lambda i: (i, 0, 0))
    g1n = pl.BlockSpec((1, 1, NB), lambda i: (i, 0, 0))
    score, label, x1, y1, x2, y2 = pl.pallas_call(
        _cand_kernel,
        grid=(G,),
        in_specs=[
            pl.BlockSpec((B, 2), lambda i: (0, 0)),
            pl.BlockSpec((1, 1, C), lambda i: (0, 0, 0)),
            pl.BlockSpec((1, 1, C), lambda i: (0, 0, 0)),
            pl.BlockSpec((B, NB, C), lambda i: (0, i, 0)),
            gbn, gbn, gbn, gbn, g1n, g1n, g1n, g1n,
        ],
        out_specs=[pl.BlockSpec((1, B, NB), lambda i: (i, 0, 0))] * 6,
        out_shape=[fld, fld, fld, fld, fld, fld],
    )(scale, clsw, mask0, pred_logits, rx, ry, rw, rh, px, py, pw, ph)

    score8 = score.reshape(-1)
    lab8 = label.reshape(-1)
    x18 = x1.reshape(-1)
    y18 = y1.reshape(-1)
    x28 = x2.reshape(-1)
    y28 = y2.reshape(-1)

    mesh = plsc.VectorSubcoreMesh(core_axis_name="c", subcore_axis_name="s")
    big = pltpu.VMEM((_NP,), jnp.float32)
    kept = pltpu.VMEM((_OUTW,), jnp.float32)
    out = functools.partial(
        pl.kernel,
        mesh=mesh,
        compiler_params=pltpu.CompilerParams(needs_layout_passes=False),
        out_type=jax.ShapeDtypeStruct((B * 6 * _OUTW,), jnp.float32),
        scratch_types=[big, big, big, big, big, big,
                       pltpu.VMEM((_NVEC,), jnp.float32),
                       pltpu.VMEM((96,), jnp.float32),
                       kept, kept, kept, kept, kept,
                       pltpu.VMEM((6 * _OUTW,), jnp.float32),
                       pltpu.SemaphoreType.DMA],
    )(_sc_nms)(score8, lab8, x18, y18, x28, y28)

    o = out.reshape(B, 6, _OUTW)[:, :, :_DETS]
    combined = jnp.stack([o[:, 0], o[:, 1], o[:, 2], o[:, 3], o[:, 4]],
                         axis=-1)
    labels = o[:, 5].astype(jnp.int32)
    return (combined, labels)


# speedup vs baseline: 330.6990x; 1.0494x over previous
import functools

import jax
import jax.numpy as jnp
from jax import lax
from jax.experimental import pallas as pl
from jax.experimental.pallas import tpu as pltpu
from jax.experimental.pallas import tpu_sc as plsc

_VAR0 = 0.1
_VAR1 = 0.2
_SCORE_T = 0.5
_NMS_T = 0.45
_DETS = 100
_MIN_SZ = 0.01
_NEG = float("-inf")

_N = 20000
_NB = 1000
_G = _N // _NB
_NP = 20480
_NVEC = _NP // 16
_NGRP = _NVEC // 16
_OUTW = 112


def _cand_kernel(scale_ref, clsw_ref, mask0_ref, logits_ref, r_ref, p_ref,
                 score_ref, label_ref, x1_ref, y1_ref, x2_ref, y2_ref):
    lg = logits_ref[...]
    m = jnp.max(lg, axis=-1, keepdims=True)
    e = jnp.exp(lg - m)
    s = jnp.sum(e, axis=-1)
    e80 = e * mask0_ref[...]
    emax = jnp.max(e80, axis=-1, keepdims=True)
    score = emax[..., 0] / s
    label = jnp.max(jnp.where(e80 == emax, clsw_ref[...], 0.0), axis=-1)

    rx = r_ref[0, 0]
    ry = r_ref[1, 0]
    rw = r_ref[2, 0]
    rh = r_ref[3, 0]
    px = p_ref[0, 0]
    py = p_ref[1, 0]
    pw = p_ref[2, 0]
    ph = p_ref[3, 0]
    cx = px + rx * _VAR0 * pw
    cy = py + ry * _VAR0 * ph
    w = pw * jnp.exp(rw * _VAR1)
    h = ph * jnp.exp(rh * _VAR1)
    tx = cx - w * 0.5
    ty = cy - h * 0.5
    bx = tx + w
    by = ty + h
    sx = scale_ref[:, 0:1]
    sy = scale_ref[:, 1:2]
    x1 = tx * sx
    y1 = ty * sy
    x2 = bx * sx
    y2 = by * sy
    ws = x2 - x1
    hs = y2 - y1
    valid = (score > _SCORE_T) & (ws >= _MIN_SZ) & (hs >= _MIN_SZ)
    score_ref[0] = jnp.where(valid, score, _NEG)
    label_ref[0] = label
    x1_ref[0] = x1
    y1_ref[0] = y1
    x2_ref[0] = x2
    y2_ref[0] = y2


_STAGES = (160, 352, 1536, 8192, _NP)


def _sc_nms(score_hbm, lab_hbm, x1_hbm, y1_hbm, x2_hbm, y2_hbm, out_hbm,
            score_v, lab_v, x1_v, y1_v, x2_v, y2_v,
            bm_v, bm2_v, kx1_v, ky1_v, kx2_v, ky2_v, karea_v, out_v, sem):
    nc = 2
    wid = lax.axis_index("s") * nc + lax.axis_index("c")
    b = wid

    @pl.when(wid < 8)
    def _():
        fields = ((score_hbm, score_v), (lab_hbm, lab_v), (x1_hbm, x1_v),
                  (y1_hbm, y1_v), (x2_hbm, x2_v), (y2_hbm, y2_v))
        handles = []
        for src, dst in fields:
            for g in range(_G):
                handles.append(pltpu.async_copy(
                    src.at[pl.ds((g * 8 + b) * _NB, _NB)],
                    dst.at[pl.ds(g * _NB, _NB)], sem))
        for h in handles:
            h.wait()

        li = lax.iota(jnp.int32, 16)
        negv = jnp.full((16,), _NEG, jnp.float32)
        zerov = jnp.zeros((16,), jnp.float32)

        for q in range((_NP - _N) // 16):
            score_v[pl.ds(_N + q * 16, 16)] = negv
            lab_v[pl.ds(_N + q * 16, 16)] = zerov
            x1_v[pl.ds(_N + q * 16, 16)] = zerov
            y1_v[pl.ds(_N + q * 16, 16)] = zerov
            x2_v[pl.ds(_N + q * 16, 16)] = zerov
            y2_v[pl.ds(_N + q * 16, 16)] = zerov

        def smax(v):
            for k in (8, 4, 2, 1):
                v = jnp.maximum(v, v[li ^ k])
            return v

        def sminv(v):
            for k in (8, 4, 2, 1):
                v = jnp.minimum(v, v[li ^ k])
            return v

        def ffsv(mask):
            return sminv(jnp.where(mask, li, jnp.int32(16)))

        def ffs(mask):
            return ffsv(mask)[0]

        for q in range(6 * _OUTW // 16):
            out_v[pl.ds(q * 16, 16)] = zerov

        bm2_v[pl.ds(_NGRP, 16)] = negv

        def group_body(g, carry):
            acc2, mx = carry
            acc16 = negv
            mxv = mx
            for jj in range(16):
                off = g * 256 + jj * 16
                sv = score_v[pl.ds(off, 16)]
                acc16 = jnp.where(li == jj, smax(sv), acc16)
                vm = sv > _NEG
                mxv = jnp.maximum(mxv, jnp.where(vm, x1_v[pl.ds(off, 16)], _NEG))
                mxv = jnp.maximum(mxv, jnp.where(vm, y1_v[pl.ds(off, 16)], _NEG))
                mxv = jnp.maximum(mxv, jnp.where(vm, x2_v[pl.ds(off, 16)], _NEG))
                mxv = jnp.maximum(mxv, jnp.where(vm, y2_v[pl.ds(off, 16)], _NEG))
            bm_v[pl.ds(g * 16, 16)] = acc16
            acc2 = jnp.where(li == (g % 16), smax(acc16), acc2)
            return acc2, mxv

        mx = negv
        for t in range(_NGRP // 16):
            acc2, mx = lax.fori_loop(t * 16, (t + 1) * 16, group_body,
                                     (negv, mx))
            bm2_v[pl.ds(t * 16, 16)] = acc2

        kcoef = smax(mx)[0] + 1.0

        def global_max():
            g = bm2_v[pl.ds(0, 16)]
            for t in range(1, 6):
                g = jnp.maximum(g, bm2_v[pl.ds(t * 16, 16)])
            return smax(g)[0]

        def body(_, carry):
            kc, gmax = carry
            alive = (kc < _DETS) & (gmax > _NEG)
            bigi = jnp.int32(1 << 30)
            g2s = bigi
            for t in range(6):
                v = bm2_v[pl.ds(t * 16, 16)]
                f = ffs(v == gmax)
                hit = f < 16
                g2s = jnp.where((g2s == bigi) & hit, t * 16 + f, g2s)
            g2 = jnp.where(alive, g2s, 0)
            bv = bm_v[pl.ds(g2 * 16, 16)]
            jfv = ffsv(bv == gmax)
            jf = jnp.where(alive, jfv[0], 0)
            j = g2 * 16 + jf
            sv = score_v[pl.ds(j * 16, 16)]
            lfv = ffsv(sv == gmax)
            lf = jnp.where(alive, lfv[0], 0)
            lmask = li == lf

            gidx = jnp.where(alive, lfv, 0)

            def extract(buf):
                return buf[pl.ds(j * 16, 16)][gidx][0]

            xv1 = extract(x1_v)
            yv1 = extract(y1_v)
            xv2 = extract(x2_v)
            yv2 = extract(y2_v)
            lv = extract(lab_v)

            sv2 = jnp.where(lmask & alive, _NEG, sv)
            score_v[pl.ds(j * 16, 16)] = sv2
            bv2 = jnp.where((li == jf) & alive, smax(sv2), bv)
            bm_v[pl.ds(g2 * 16, 16)] = bv2
            t2 = g2 // 16
            b2v = bm2_v[pl.ds(t2 * 16, 16)]
            b2v2 = jnp.where((li == (g2 % 16)) & alive, smax(bv2), b2v)
            bm2_v[pl.ds(t2 * 16, 16)] = b2v2

            offs = lv * kcoef
            cx1 = xv1 + offs
            cy1 = yv1 + offs
            cx2 = xv2 + offs
            cy2 = yv2 + offs
            carea = (cx2 - cx1) * (cy2 - cy1)

            def kbody(q, rej):
                lanev = q * 16 + li
                xx1 = jnp.maximum(cx1, kx1_v[pl.ds(q * 16, 16)])
                yy1 = jnp.maximum(cy1, ky1_v[pl.ds(q * 16, 16)])
                xx2 = jnp.minimum(cx2, kx2_v[pl.ds(q * 16, 16)])
                yy2 = jnp.minimum(cy2, ky2_v[pl.ds(q * 16, 16)])
                w = jnp.maximum(0.0, xx2 - xx1)
                h = jnp.maximum(0.0, yy2 - yy1)
                inter = w * h
                iou = inter / (carea + karea_v[pl.ds(q * 16, 16)] - inter
                               + 1e-12)
                m = (iou > _NMS_T) & (lanev < kc)
                return rej | jnp.where(ffs(m) < 16, jnp.int32(1), jnp.int32(0))

            nk = (kc + 15) // 16
            rej = lax.fori_loop(0, nk, kbody, jnp.int32(0))
            acc = (rej == 0) & alive

            q = kc // 16
            l = kc % 16
            am = acc & (li == l)

            def ins(buf, base, val):
                cur = buf[pl.ds(base + q * 16, 16)]
                buf[pl.ds(base + q * 16, 16)] = jnp.where(am, val, cur)

            ins(kx1_v, 0, cx1)
            ins(ky1_v, 0, cy1)
            ins(kx2_v, 0, cx2)
            ins(ky2_v, 0, cy2)
            ins(karea_v, 0, carea)
            ins(out_v, 0 * _OUTW, xv1)
            ins(out_v, 1 * _OUTW, yv1)
            ins(out_v, 2 * _OUTW, xv2)
            ins(out_v, 3 * _OUTW, yv2)
            ins(out_v, 4 * _OUTW, gmax)
            ins(out_v, 5 * _OUTW, lv)

            kc2 = kc + jnp.where(acc, 1, 0)
            return kc2, global_max()

        carry = (jnp.int32(0), global_max())
        for cap in _STAGES:
            kc, gmax = carry
            n = jnp.where((kc < _DETS) & (gmax > _NEG), cap, 0)
            carry = lax.fori_loop(0, n, body, carry)
        pltpu.sync_copy(out_v, out_hbm.at[pl.ds(b * (6 * _OUTW), 6 * _OUTW)])


def kernel(pred_logits, pred_boxes, priors, target_sizes):
    B, N, C = pred_logits.shape
    NB = 1000
    G = N // NB
    ts = target_sizes.astype(jnp.float32)
    scale = jnp.stack([ts[:, 1], ts[:, 0]], axis=1)

    rs = jnp.transpose(pred_boxes.reshape(B, G, NB, 4),
                       (3, 1, 0, 2))
    ps = jnp.transpose(priors.reshape(G, NB, 4),
                       (2, 0, 1)).reshape(4, G, 1, NB)

    clsw = jnp.arange(0, C, dtype=jnp.float32).reshape(1, 1, C)
    mask0 = jnp.concatenate([jnp.zeros((1,), jnp.float32),
                             jnp.ones((C - 1,), jnp.float32)]
                            ).reshape(1, 1, C)

    fld = jax.ShapeDtypeStruct((G, B, NB), jnp.float32)
    score, label, x1, y1, x2, y2 = pl.pallas_call(
        _cand_kernel,
        grid=(G,),
        in_specs=[
            pl.BlockSpec((B, 2), lambda i: (0, 0)),
            pl.BlockSpec((1, 1, C), lambda i: (0, 0, 0)),
            pl.BlockSpec((1, 1, C), lambda i: (0, 0, 0)),
            pl.BlockSpec((B, NB, C), lambda i: (0, i, 0)),
            pl.BlockSpec((4, 1, B, NB), lambda i: (0, i, 0, 0)),
            pl.BlockSpec((4, 1, 1, NB), lambda i: (0, i, 0, 0)),
        ],
        out_specs=[pl.BlockSpec((1, B, NB), lambda i: (i, 0, 0))] * 6,
        out_shape=[fld, fld, fld, fld, fld, fld],
    )(scale, clsw, mask0, pred_logits, rs, ps)

    score8 = score.reshape(-1)
    lab8 = label.reshape(-1)
    x18 = x1.reshape(-1)
    y18 = y1.reshape(-1)
    x28 = x2.reshape(-1)
    y28 = y2.reshape(-1)

    mesh = plsc.VectorSubcoreMesh(core_axis_name="c", subcore_axis_name="s")
    big = pltpu.VMEM((_NP,), jnp.float32)
    kept = pltpu.VMEM((_OUTW,), jnp.float32)
    out = functools.partial(
        pl.kernel,
        mesh=mesh,
        compiler_params=pltpu.CompilerParams(needs_layout_passes=False),
        out_type=jax.ShapeDtypeStruct((B * 6 * _OUTW,), jnp.float32),
        scratch_types=[big, big, big, big, big, big,
                       pltpu.VMEM((_NVEC,), jnp.float32),
                       pltpu.VMEM((96,), jnp.float32),
                       kept, kept, kept, kept, kept,
                       pltpu.VMEM((6 * _OUTW,), jnp.float32),
                       pltpu.SemaphoreType.DMA],
    )(_sc_nms)(score8, lab8, x18, y18, x28, y28)

    o = out.reshape(B, 6, _OUTW)[:, :, :_DETS]
    combined = jnp.stack([o[:, 0], o[:, 1], o[:, 2], o[:, 3], o[:, 4]],
                         axis=-1)
    labels = o[:, 5].astype(jnp.int32)
    return (combined, labels)


# speedup vs baseline: 337.7916x; 1.0214x over previous
import functools

import jax
import jax.numpy as jnp
from jax import lax
from jax.experimental import pallas as pl
from jax.experimental.pallas import tpu as pltpu
from jax.experimental.pallas import tpu_sc as plsc

_VAR0 = 0.1
_VAR1 = 0.2
_SCORE_T = 0.5
_NMS_T = 0.45
_DETS = 100
_MIN_SZ = 0.01
_NEG = float("-inf")

_N = 20000
_NB = 1000
_G = _N // _NB
_NP = 20480
_NVEC = _NP // 16
_NGRP = _NVEC // 16
_OUTW = 112


def _cand_kernel(scale_ref, clsw_ref, mask0_ref, logits_ref, r_ref, p_ref,
                 score_ref, label_ref, x1_ref, y1_ref, x2_ref, y2_ref):
    lg = logits_ref[...]
    m = jnp.max(lg, axis=-1, keepdims=True)
    e = jnp.exp(lg - m)
    s = jnp.sum(e, axis=-1)
    e80 = e * mask0_ref[...]
    emax = jnp.max(e80, axis=-1, keepdims=True)
    score = emax[..., 0] / s
    label = jnp.max(jnp.where(e80 == emax, clsw_ref[...], 0.0), axis=-1)

    rx = r_ref[0, 0]
    ry = r_ref[1, 0]
    rw = r_ref[2, 0]
    rh = r_ref[3, 0]
    px = p_ref[0, 0]
    py = p_ref[1, 0]
    pw = p_ref[2, 0]
    ph = p_ref[3, 0]
    cx = px + rx * _VAR0 * pw
    cy = py + ry * _VAR0 * ph
    w = pw * jnp.exp(rw * _VAR1)
    h = ph * jnp.exp(rh * _VAR1)
    tx = cx - w * 0.5
    ty = cy - h * 0.5
    bx = tx + w
    by = ty + h
    sx = scale_ref[:, 0:1]
    sy = scale_ref[:, 1:2]
    x1 = tx * sx
    y1 = ty * sy
    x2 = bx * sx
    y2 = by * sy
    ws = x2 - x1
    hs = y2 - y1
    valid = (score > _SCORE_T) & (ws >= _MIN_SZ) & (hs >= _MIN_SZ)
    score_ref[0] = jnp.where(valid, score, _NEG)
    label_ref[0] = label
    x1_ref[0] = x1
    y1_ref[0] = y1
    x2_ref[0] = x2
    y2_ref[0] = y2


_STAGES = (128, 128, 1024, 4096, 16384)


def _sc_nms(score_hbm, lab_hbm, x1_hbm, y1_hbm, x2_hbm, y2_hbm, out_hbm,
            score_v, lab_v, x1_v, y1_v, x2_v, y2_v,
            bm_v, bm2_v, kx1_v, ky1_v, kx2_v, ky2_v, karea_v, out_v, sem):
    nc = 2
    wid = lax.axis_index("s") * nc + lax.axis_index("c")
    b = wid

    @pl.when(wid < 8)
    def _():
        fields = ((score_hbm, score_v), (lab_hbm, lab_v), (x1_hbm, x1_v),
                  (y1_hbm, y1_v), (x2_hbm, x2_v), (y2_hbm, y2_v))
        handles = []
        for src, dst in fields:
            for g in range(_G):
                handles.append(pltpu.async_copy(
                    src.at[pl.ds((g * 8 + b) * _NB, _NB)],
                    dst.at[pl.ds(g * _NB, _NB)], sem))
        for h in handles:
            h.wait()

        li = lax.iota(jnp.int32, 16)
        negv = jnp.full((16,), _NEG, jnp.float32)
        zerov = jnp.zeros((16,), jnp.float32)

        for q in range((_NP - _N) // 16):
            score_v[pl.ds(_N + q * 16, 16)] = negv
            lab_v[pl.ds(_N + q * 16, 16)] = zerov
            x1_v[pl.ds(_N + q * 16, 16)] = zerov
            y1_v[pl.ds(_N + q * 16, 16)] = zerov
            x2_v[pl.ds(_N + q * 16, 16)] = zerov
            y2_v[pl.ds(_N + q * 16, 16)] = zerov

        def smax(v):
            for k in (8, 4, 2, 1):
                v = jnp.maximum(v, v[li ^ k])
            return v

        def sminv(v):
            for k in (8, 4, 2, 1):
                v = jnp.minimum(v, v[li ^ k])
            return v

        def ffsv(mask):
            return sminv(jnp.where(mask, li, jnp.int32(16)))

        def ffs(mask):
            return ffsv(mask)[0]

        for q in range(6 * _OUTW // 16):
            out_v[pl.ds(q * 16, 16)] = zerov

        bm2_v[pl.ds(_NGRP, 16)] = negv

        def group_body(g, carry):
            acc2, mx = carry
            acc16 = negv
            mxv = mx
            for jj in range(16):
                off = g * 256 + jj * 16
                sv = score_v[pl.ds(off, 16)]
                acc16 = jnp.where(li == jj, smax(sv), acc16)
                vm = sv > _NEG
                mxv = jnp.maximum(mxv, jnp.where(vm, x1_v[pl.ds(off, 16)], _NEG))
                mxv = jnp.maximum(mxv, jnp.where(vm, y1_v[pl.ds(off, 16)], _NEG))
                mxv = jnp.maximum(mxv, jnp.where(vm, x2_v[pl.ds(off, 16)], _NEG))
                mxv = jnp.maximum(mxv, jnp.where(vm, y2_v[pl.ds(off, 16)], _NEG))
            bm_v[pl.ds(g * 16, 16)] = acc16
            acc2 = jnp.where(li == (g % 16), smax(acc16), acc2)
            return acc2, mxv

        mx = negv
        for t in range(_NGRP // 16):
            acc2, mx = lax.fori_loop(t * 16, (t + 1) * 16, group_body,
                                     (negv, mx))
            bm2_v[pl.ds(t * 16, 16)] = acc2

        kcoef = smax(mx)[0] + 1.0

        def global_max():
            g = bm2_v[pl.ds(0, 16)]
            for t in range(1, 6):
                g = jnp.maximum(g, bm2_v[pl.ds(t * 16, 16)])
            return smax(g)[0]

        def body(_, carry):
            kc, gmax = carry
            alive = (kc < _DETS) & (gmax > _NEG)
            bigi = jnp.int32(1 << 30)
            g2s = bigi
            for t in range(6):
                v = bm2_v[pl.ds(t * 16, 16)]
                f = ffs(v == gmax)
                hit = f < 16
                g2s = jnp.where((g2s == bigi) & hit, t * 16 + f, g2s)
            g2 = jnp.where(alive, g2s, 0)
            bv = bm_v[pl.ds(g2 * 16, 16)]
            jfv = ffsv(bv == gmax)
            jf = jnp.where(alive, jfv[0], 0)
            j = g2 * 16 + jf
            sv = score_v[pl.ds(j * 16, 16)]
            lfv = ffsv(sv == gmax)
            lf = jnp.where(alive, lfv[0], 0)
            lmask = li == lf

            gidx = jnp.where(alive, lfv, 0)

            def extract(buf):
                return buf[pl.ds(j * 16, 16)][gidx][0]

            xv1 = extract(x1_v)
            yv1 = extract(y1_v)
            xv2 = extract(x2_v)
            yv2 = extract(y2_v)
            lv = extract(lab_v)

            sv2 = jnp.where(lmask & alive, _NEG, sv)
            score_v[pl.ds(j * 16, 16)] = sv2
            bv2 = jnp.where((li == jf) & alive, smax(sv2), bv)
            bm_v[pl.ds(g2 * 16, 16)] = bv2
            t2 = g2 // 16
            b2v = bm2_v[pl.ds(t2 * 16, 16)]
            b2v2 = jnp.where((li == (g2 % 16)) & alive, smax(bv2), b2v)
            bm2_v[pl.ds(t2 * 16, 16)] = b2v2

            offs = lv * kcoef
            cx1 = xv1 + offs
            cy1 = yv1 + offs
            cx2 = xv2 + offs
            cy2 = yv2 + offs
            carea = (cx2 - cx1) * (cy2 - cy1)

            def kbody(q, rej):
                lanev = q * 16 + li
                xx1 = jnp.maximum(cx1, kx1_v[pl.ds(q * 16, 16)])
                yy1 = jnp.maximum(cy1, ky1_v[pl.ds(q * 16, 16)])
                xx2 = jnp.minimum(cx2, kx2_v[pl.ds(q * 16, 16)])
                yy2 = jnp.minimum(cy2, ky2_v[pl.ds(q * 16, 16)])
                w = jnp.maximum(0.0, xx2 - xx1)
                h = jnp.maximum(0.0, yy2 - yy1)
                inter = w * h
                iou = inter / (carea + karea_v[pl.ds(q * 16, 16)] - inter
                               + 1e-12)
                m = (iou > _NMS_T) & (lanev < kc)
                return rej | jnp.where(ffs(m) < 16, jnp.int32(1), jnp.int32(0))

            nk = (kc + 15) // 16
            rej = lax.fori_loop(0, nk, kbody, jnp.int32(0))
            acc = (rej == 0) & alive

            q = kc // 16
            l = kc % 16
            am = acc & (li == l)

            def ins(buf, base, val):
                cur = buf[pl.ds(base + q * 16, 16)]
                buf[pl.ds(base + q * 16, 16)] = jnp.where(am, val, cur)

            ins(kx1_v, 0, cx1)
            ins(ky1_v, 0, cy1)
            ins(kx2_v, 0, cx2)
            ins(ky2_v, 0, cy2)
            ins(karea_v, 0, carea)
            ins(out_v, 0 * _OUTW, xv1)
            ins(out_v, 1 * _OUTW, yv1)
            ins(out_v, 2 * _OUTW, xv2)
            ins(out_v, 3 * _OUTW, yv2)
            ins(out_v, 4 * _OUTW, gmax)
            ins(out_v, 5 * _OUTW, lv)

            kc2 = kc + jnp.where(acc, 1, 0)
            return kc2, global_max()

        carry = (jnp.int32(0), global_max())
        for cap in _STAGES:
            kc, gmax = carry
            n = jnp.where((kc < _DETS) & (gmax > _NEG), cap, 0)
            carry = lax.fori_loop(0, n, body, carry)
        pltpu.sync_copy(out_v, out_hbm.at[pl.ds(b * (6 * _OUTW), 6 * _OUTW)])


def kernel(pred_logits, pred_boxes, priors, target_sizes):
    B, N, C = pred_logits.shape
    NB = 1000
    G = N // NB
    ts = target_sizes.astype(jnp.float32)
    scale = jnp.stack([ts[:, 1], ts[:, 0]], axis=1)

    rs = jnp.transpose(pred_boxes.reshape(B, G, NB, 4),
                       (3, 1, 0, 2))
    ps = jnp.transpose(priors.reshape(G, NB, 4),
                       (2, 0, 1)).reshape(4, G, 1, NB)

    clsw = jnp.arange(0, C, dtype=jnp.float32).reshape(1, 1, C)
    mask0 = jnp.concatenate([jnp.zeros((1,), jnp.float32),
                             jnp.ones((C - 1,), jnp.float32)]
                            ).reshape(1, 1, C)

    fld = jax.ShapeDtypeStruct((G, B, NB), jnp.float32)
    score, label, x1, y1, x2, y2 = pl.pallas_call(
        _cand_kernel,
        grid=(G,),
        in_specs=[
            pl.BlockSpec((B, 2), lambda i: (0, 0)),
            pl.BlockSpec((1, 1, C), lambda i: (0, 0, 0)),
            pl.BlockSpec((1, 1, C), lambda i: (0, 0, 0)),
            pl.BlockSpec((B, NB, C), lambda i: (0, i, 0)),
            pl.BlockSpec((4, 1, B, NB), lambda i: (0, i, 0, 0)),
            pl.BlockSpec((4, 1, 1, NB), lambda i: (0, i, 0, 0)),
        ],
        out_specs=[pl.BlockSpec((1, B, NB), lambda i: (i, 0, 0))] * 6,
        out_shape=[fld, fld, fld, fld, fld, fld],
    )(scale, clsw, mask0, pred_logits, rs, ps)

    score8 = score.reshape(-1)
    lab8 = label.reshape(-1)
    x18 = x1.reshape(-1)
    y18 = y1.reshape(-1)
    x28 = x2.reshape(-1)
    y28 = y2.reshape(-1)

    mesh = plsc.VectorSubcoreMesh(core_axis_name="c", subcore_axis_name="s")
    big = pltpu.VMEM((_NP,), jnp.float32)
    kept = pltpu.VMEM((_OUTW,), jnp.float32)
    out = functools.partial(
        pl.kernel,
        mesh=mesh,
        compiler_params=pltpu.CompilerParams(needs_layout_passes=False),
        out_type=jax.ShapeDtypeStruct((B * 6 * _OUTW,), jnp.float32),
        scratch_types=[big, big, big, big, big, big,
                       pltpu.VMEM((_NVEC,), jnp.float32),
                       pltpu.VMEM((96,), jnp.float32),
                       kept, kept, kept, kept, kept,
                       pltpu.VMEM((6 * _OUTW,), jnp.float32),
                       pltpu.SemaphoreType.DMA],
    )(_sc_nms)(score8, lab8, x18, y18, x28, y28)

    o = out.reshape(B, 6, _OUTW)[:, :, :_DETS]
    combined = jnp.stack([o[:, 0], o[:, 1], o[:, 2], o[:, 3], o[:, 4]],
                         axis=-1)
    labels = o[:, 5].astype(jnp.int32)
    return (combined, labels)
